# baseline (speedup 1.0000x reference)
import functools

import jax
import jax.numpy as jnp
from jax import lax
from jax.experimental import pallas as pl
from jax.experimental.pallas import tpu as pltpu
from jax.experimental.pallas import tpu_sc as plsc

_D = 1024
_MAX_LEN = 32768
_SEQ = 8192
_NC = 2
_NS = 16
_NW = _NC * _NS
_B_PER_W = _SEQ // _NW
_C = 16
_PAIR = 2
_SC = _C * _PAIR
_NP = _B_PER_W // _SC
_PBUF = 3
_P_AHEAD = 2
_P_SOUT = _PBUF - _P_AHEAD

_mesh = plsc.VectorSubcoreMesh(core_axis_name="c", subcore_axis_name="s")


@functools.partial(
    pl.kernel,
    mesh=_mesh,
    out_type=jax.ShapeDtypeStruct((_SEQ, _D), jnp.float32),
    scratch_types=(
        [pltpu.VMEM((_B_PER_W,), jnp.int32),
         pltpu.VMEM((_PBUF, _SC, _D), jnp.float32)]
        + [pltpu.SemaphoreType.DMA] * ((_PAIR + 1) * _PBUF)
    ),
)
def _gather(table_hbm, idx_hbm, out_hbm, idx_v, bufs, *sems):
    cid = lax.axis_index("c")
    sid = lax.axis_index("s")
    wid = sid * _NC + cid
    base = wid * _B_PER_W
    gsem = sems[: _PAIR * _PBUF]
    ssem = sems[_PAIR * _PBUF :]
    pltpu.sync_copy(idx_hbm.at[pl.ds(base, _B_PER_W)], idx_v)

    def start_gathers(p):
        b = p % _PBUF
        return [
            pltpu.async_copy(
                table_hbm.at[idx_v.at[pl.ds(p * _SC + h * _C, _C)]],
                bufs.at[b, pl.ds(h * _C, _C)],
                gsem[b * _PAIR + h],
            )
            for h in range(_PAIR)
        ]

    def start_store(p):
        b = p % _PBUF
        return pltpu.async_copy(
            bufs.at[b], out_hbm.at[pl.ds(base + p * _SC, _SC)], ssem[b]
        )

    gathers = [None] * _NP
    stores = [None] * _NP
    for p in range(_P_AHEAD):
        gathers[p] = start_gathers(p)
    for p in range(_NP):
        if p >= _P_SOUT:
            stores[p - _P_SOUT].wait()
        if p + _P_AHEAD < _NP:
            gathers[p + _P_AHEAD] = start_gathers(p + _P_AHEAD)
        for g in gathers[p]:
            g.wait()
        stores[p] = start_store(p)
    for p in range(_NP - _P_SOUT, _NP):
        stores[p].wait()


def kernel(position_ids, PosEnc):
    table = PosEnc.reshape(_MAX_LEN, _D)
    idx = position_ids.astype(jnp.int32)
    out = _gather(table, idx)
    return out.reshape(1, _SEQ, _D)
